# baseline (speedup 1.0000x reference)
import jax
import jax.numpy as jnp
from jax.experimental import pallas as pl
from jax.experimental.pallas import tpu as pltpu

_BM = 256
_DEPTH = 2


def _body(x_ref, a_hbm, o_ref, abuf0, abuf1, sem0, sem1):
    i = pl.program_id(0)
    steps = pl.num_programs(0)
    nbuf = 2 * _DEPTH

    @pl.when(i == 0)
    def _prologue():
        for j in range(nbuf):
            buf, sem = (abuf0, sem0) if j % 2 == 0 else (abuf1, sem1)
            pltpu.make_async_copy(
                a_hbm.at[pl.ds(j * _BM, _BM), :], buf.at[j // 2], sem.at[j // 2]
            ).start()

    slot = jax.lax.rem(jax.lax.div(i, 2), _DEPTH)

    @pl.when(jax.lax.rem(i, 2) == 0)
    def _even():
        pltpu.make_async_copy(
            a_hbm.at[pl.ds(i * _BM, _BM), :], abuf0.at[slot], sem0.at[slot]
        ).wait()
        t = jnp.dot(abuf0[slot], x_ref[...], preferred_element_type=jnp.float32)
        o_ref[...] = (t > 0.5).astype(jnp.float32)

        @pl.when(i + nbuf < steps)
        def _():
            nxt = i + nbuf
            pltpu.make_async_copy(
                a_hbm.at[pl.ds(nxt * _BM, _BM), :], abuf0.at[slot], sem0.at[slot]
            ).start()

    @pl.when(jax.lax.rem(i, 2) == 1)
    def _odd():
        pltpu.make_async_copy(
            a_hbm.at[pl.ds(i * _BM, _BM), :], abuf1.at[slot], sem1.at[slot]
        ).wait()
        t = jnp.dot(abuf1[slot], x_ref[...], preferred_element_type=jnp.float32)
        o_ref[...] = (t > 0.5).astype(jnp.float32)

        @pl.when(i + nbuf < steps)
        def _():
            nxt = i + nbuf
            pltpu.make_async_copy(
                a_hbm.at[pl.ds(nxt * _BM, _BM), :], abuf1.at[slot], sem1.at[slot]
            ).start()


def kernel(x, a):
    m, k = a.shape
    n = x.shape[1]
    return pl.pallas_call(
        _body,
        grid=(m // _BM,),
        in_specs=[
            pl.BlockSpec((k, n), lambda i: (0, 0)),
            pl.BlockSpec(memory_space=pltpu.MemorySpace.HBM),
        ],
        out_specs=pl.BlockSpec((_BM, n), lambda i: (i, 0)),
        out_shape=jax.ShapeDtypeStruct((m, n), jnp.float32),
        scratch_shapes=[
            pltpu.VMEM((_DEPTH, _BM, 8192), jnp.float32),
            pltpu.VMEM((_DEPTH, _BM, 8192), jnp.float32),
            pltpu.SemaphoreType.DMA((_DEPTH,)),
            pltpu.SemaphoreType.DMA((_DEPTH,)),
        ],
        compiler_params=pltpu.CompilerParams(
            dimension_semantics=("arbitrary",),
        ),
    )(x, a)
